# Initial kernel scaffold; baseline (speedup 1.0000x reference)
import jax
import jax.numpy as jnp
from jax.experimental import pallas as pl


def kernel(x, edge_index_rel0, edge_index_rel1, W1_0, al1_0, ar1_0, resW1_0, b1_0, W1_1, al1_1, ar1_1, resW1_1, b1_1, W2_0, al2_0, ar2_0, resW2_0, b2_0, W2_1, al2_1, ar2_1, resW2_1, b2_1):
    raise NotImplementedError("write your pallas kernel here")


# speedup vs baseline: 34.5703x; 34.5703x over previous
import functools

import jax
import jax.numpy as jnp
from jax import lax
from jax.experimental import pallas as pl
from jax.experimental.pallas import tpu as pltpu
from jax.experimental.pallas import tpu_sc as plsc

N = 10000
E = 320000
H = 4

_BM = 1000
_NSUB = 16
_KA = 128
_KB = 64
_CHA = E // _KA
_CHB = E // _KB


def _featT_body(x_ref, w_ref, o_ref):
    o_ref[...] = jnp.dot(x_ref[...], w_ref[...], preferred_element_type=jnp.float32)


@functools.partial(jax.jit, static_argnames=("din",))
def _featT(x, Wcat, din):
    return pl.pallas_call(
        _featT_body,
        grid=(4, N // _BM),
        in_specs=[
            pl.BlockSpec((_BM, din), lambda rc, i: (i, 0)),
            pl.BlockSpec((din, 128), lambda rc, i: (0, rc)),
        ],
        out_specs=pl.BlockSpec((_BM, 128), lambda rc, i: (rc * (N // _BM) + i, 0)),
        out_shape=jax.ShapeDtypeStruct((4 * N, 128), jnp.float32),
    )(x, Wcat)


def _aux_body(x_ref, w_ref, rw_ref, ab0_ref, ab1_ref, brow_ref,
              res_ref, eler_ref):
    x = x_ref[...]
    feat = jnp.dot(x, w_ref[...], preferred_element_type=jnp.float32)
    res_ref[...] = jnp.dot(x, rw_ref[...], preferred_element_type=jnp.float32) + brow_ref[...]
    t0 = jnp.dot(feat, ab0_ref[...], preferred_element_type=jnp.float32)
    t1 = jnp.dot(feat, ab1_ref[...], preferred_element_type=jnp.float32)
    eler_ref[...] = jnp.stack([t0, t1])


@functools.partial(jax.jit, static_argnames=("din",))
def _aux(x, Wcat, resWcat, AB0, AB1, brow, din):
    return pl.pallas_call(
        _aux_body,
        grid=(N // _BM,),
        in_specs=[
            pl.BlockSpec((_BM, din), lambda i: (i, 0)),
            pl.BlockSpec((din, 512), lambda i: (0, 0)),
            pl.BlockSpec((din, 512), lambda i: (0, 0)),
            pl.BlockSpec((512, 8), lambda i: (0, 0)),
            pl.BlockSpec((512, 8), lambda i: (0, 0)),
            pl.BlockSpec((1, 512), lambda i: (0, 0)),
        ],
        out_specs=[
            pl.BlockSpec((_BM, 512), lambda i: (i, 0)),
            pl.BlockSpec((2, _BM, 8), lambda i: (0, i, 0)),
        ],
        out_shape=[
            jax.ShapeDtypeStruct((N, 512), jnp.float32),
            jax.ShapeDtypeStruct((2, N, 8), jnp.float32),
        ],
    )(x, Wcat, resWcat, AB0, AB1, brow)


def _scores_body(src0, dst0, src1, dst1, elerT, z4,
                 denP, eeT,
                 srcb, dstb, eeb, tab, den):
    cid = lax.axis_index("c")
    sid = lax.axis_index("s")
    wid = 2 * sid + cid
    lanes16 = lax.iota(jnp.int32, 16) * 16
    srcs = (src0, src1)
    dsts = (dst0, dst1)

    for r in (0, 1):
        src, dst = srcs[r], dsts[r]
        pltpu.sync_copy(elerT.at[pl.ds(r * 8 * N, 8 * N)], tab)
        pltpu.sync_copy(z4, den)

        def body_a(j, _):
            k = wid + 32 * j
            off = k * _KA
            pltpu.sync_copy(src.at[pl.ds(off, _KA)], srcb)
            pltpu.sync_copy(dst.at[pl.ds(off, _KA)], dstb)
            for g in range(_KA // 16):
                sv8 = srcb[pl.ds(16 * g, 16)] * 8
                dv = dstb[pl.ds(16 * g, 16)]
                dv8 = dv * 8
                dv4 = dv * 4
                rows = 256 * g + lanes16
                for h in range(H):
                    el = plsc.load_gather(tab, [sv8 + h])
                    er = plsc.load_gather(tab, [dv8 + (4 + h)])
                    v = el + er
                    v = jnp.maximum(v, 0.2 * v)
                    v = jnp.exp(v)
                    plsc.store_scatter(eeb, [rows + h], v)
                    plsc.addupdate_scatter(den, [dv4 + h], v)
            pltpu.sync_copy(eeb, eeT.at[pl.ds((r * E + off) * 16, _KA * 16)])
            return 0

        nj = (_CHA - wid + 31) // 32
        lax.fori_loop(0, nj, body_a, 0)
        pltpu.sync_copy(den, denP.at[pl.ds((r * 32 + wid) * 4 * N, 4 * N)])


def _scores(src0, dst0, src1, dst1, elerT, z4):
    mesh = plsc.VectorSubcoreMesh(core_axis_name="c", subcore_axis_name="s",
                                  num_cores=2, num_subcores=_NSUB)
    f = pl.kernel(
        _scores_body,
        out_type=[
            jax.ShapeDtypeStruct((256 * N,), jnp.float32),
            jax.ShapeDtypeStruct((32 * E,), jnp.float32),
        ],
        mesh=mesh,
        compiler_params=pltpu.CompilerParams(needs_layout_passes=False),
        scratch_types=[
            pltpu.VMEM((_KA,), jnp.int32),
            pltpu.VMEM((_KA,), jnp.int32),
            pltpu.VMEM((16 * _KA,), jnp.float32),
            pltpu.VMEM((8 * N,), jnp.float32),
            pltpu.VMEM((4 * N,), jnp.float32),
        ],
    )
    return f(src0, dst0, src1, dst1, elerT, z4)


def _dred_body(p_ref, o_ref):
    o_ref[...] = jnp.sum(p_ref[...], axis=1)


def _dred(denP):
    return pl.pallas_call(
        _dred_body,
        grid=(1,),
        in_specs=[pl.BlockSpec((2, 32, 4 * N), lambda i: (0, 0, 0))],
        out_specs=pl.BlockSpec((2, 4 * N), lambda i: (0, 0)),
        out_shape=jax.ShapeDtypeStruct((2, 4 * N), jnp.float32),
    )(denP)


def _agg_body(src0, dst0, src1, dst1, featT, den2, eeT, z128,
              rstT,
              srcb, dstb, adjb, featb, eech, den,
              out_sp):
    cid = lax.axis_index("c")
    sid = lax.axis_index("s")
    lanes16 = lax.iota(jnp.int32, 16) * 16
    c0 = 2 * cid
    srcs = (src0, src1)
    dsts = (dst0, dst1)

    for r in (0, 1):
        src, dst = srcs[r], dsts[r]
        pltpu.sync_copy(den2.at[pl.ds(r * 4 * N, 4 * N)], den)

        @pl.when(sid < _NSUB - 1)
        def _():
            pltpu.sync_copy(z128.at[pl.ds(sid * 632, 632)],
                            out_sp.at[pl.ds(sid * 632, 632)])

        @pl.when(sid == _NSUB - 1)
        def _():
            pltpu.sync_copy(z128.at[pl.ds(9480, 520)],
                            out_sp.at[pl.ds(9480, 520)])
        plsc.subcore_barrier()

        def body_b(j, _):
            k = sid + _NSUB * j
            off = k * _KB
            pltpu.sync_copy(src.at[pl.ds(off, _KB)], srcb)
            pltpu.sync_copy(dst.at[pl.ds(off, _KB)], dstb)
            base = (2 * r + cid) * N
            for g in range(_KB // 16):
                adjb[pl.ds(16 * g, 16)] = srcb[pl.ds(16 * g, 16)] + base
            pltpu.sync_copy(featT.at[adjb], featb)
            pltpu.sync_copy(eeT.at[pl.ds((r * E + off) * 16, _KB * 16)], eech)
            for g in range(_KB // 16):
                dv4 = dstb[pl.ds(16 * g, 16)] * 4
                rows = 256 * g + lanes16
                d0 = plsc.load_gather(den, [dv4 + c0])
                d1 = plsc.load_gather(den, [dv4 + c0 + 1])
                e0 = plsc.load_gather(eech, [rows + c0])
                e1 = plsc.load_gather(eech, [rows + c0 + 1])
                a0 = e0 / jnp.maximum(d0, 1e-9)
                a1 = e1 / jnp.maximum(d1, 1e-9)
                for i in range(16):
                    e = 16 * g + i
                    va0 = jnp.full((16,), a0[i])
                    va1 = jnp.full((16,), a1[i])
                    for q in range(8):
                        va = va0 if q < 4 else va1
                        featb[e, pl.ds(16 * q, 16)] = featb[e, pl.ds(16 * q, 16)] * va
            pltpu.sync_copy(featb, out_sp.at[dstb], add=True)
            return 0

        nj = (_CHB - sid + _NSUB - 1) // _NSUB
        lax.fori_loop(0, nj, body_b, 0)
        plsc.subcore_barrier()

        @pl.when(sid < _NSUB - 1)
        def _():
            pltpu.sync_copy(out_sp.at[pl.ds(sid * 632, 632)],
                            rstT.at[pl.ds((2 * r + cid) * N + sid * 632, 632)])

        @pl.when(sid == _NSUB - 1)
        def _():
            pltpu.sync_copy(out_sp.at[pl.ds(9480, 520)],
                            rstT.at[pl.ds((2 * r + cid) * N + 9480, 520)])
        plsc.subcore_barrier()


def _agg(src0, dst0, src1, dst1, featT, den2, eeT, z128):
    mesh = plsc.VectorSubcoreMesh(core_axis_name="c", subcore_axis_name="s",
                                  num_cores=2, num_subcores=_NSUB)
    f = pl.kernel(
        _agg_body,
        out_type=jax.ShapeDtypeStruct((4 * N, 128), jnp.float32),
        mesh=mesh,
        compiler_params=pltpu.CompilerParams(needs_layout_passes=False),
        scratch_types=[
            pltpu.VMEM((_KB,), jnp.int32),
            pltpu.VMEM((_KB,), jnp.int32),
            pltpu.VMEM((_KB,), jnp.int32),
            pltpu.VMEM((_KB, 128), jnp.float32),
            pltpu.VMEM((16 * _KB,), jnp.float32),
            pltpu.VMEM((4 * N,), jnp.float32),
            pltpu.VMEM_SHARED((N, 128), jnp.float32),
        ],
    )
    return f(src0, dst0, src1, dst1, featT, den2, eeT, z128)


def _combine_body(rst_ref, res_ref, o_ref, *, act):
    r00 = rst_ref[0, 0]
    r01 = rst_ref[0, 1]
    r10 = rst_ref[1, 0]
    r11 = rst_ref[1, 1]
    h0 = jnp.concatenate([r00, r01], axis=1) + res_ref[:, 0:256]
    h1 = jnp.concatenate([r10, r11], axis=1) + res_ref[:, 256:512]
    v = 0.5 * (h0 + h1)
    if act:
        v = jnp.where(v > 0, v, jnp.exp(jnp.minimum(v, 0.0)) - 1.0)
    o_ref[...] = v


@functools.partial(jax.jit, static_argnames=("act",))
def _combine(rstT, rescat, act):
    body = functools.partial(_combine_body, act=act)
    return pl.pallas_call(
        body,
        grid=(N // _BM,),
        in_specs=[
            pl.BlockSpec((2, 2, _BM, 128), lambda i: (0, 0, i, 0)),
            pl.BlockSpec((_BM, 512), lambda i: (i, 0)),
        ],
        out_specs=pl.BlockSpec((_BM, 256), lambda i: (i, 0)),
        out_shape=jax.ShapeDtypeStruct((N, 256), jnp.float32),
    )(rstT, rescat)


def _build_ab(al, ar, r):
    ab = jnp.zeros((512, 8), jnp.float32)
    for h in range(H):
        rows = slice(r * 256 + h * 64, r * 256 + (h + 1) * 64)
        ab = ab.at[rows, h].set(al[h])
        ab = ab.at[rows, 4 + h].set(ar[h])
    return ab


def _layer(x, src0, dst0, src1, dst1, W_0, resW_0, b_0, W_1, resW_1, b_1,
           al_0, ar_0, al_1, ar_1, din, act):
    Wcat = jnp.concatenate([W_0, W_1], axis=1)
    rWcat = jnp.concatenate([resW_0, resW_1], axis=1)
    AB0 = _build_ab(al_0, ar_0, 0)
    AB1 = _build_ab(al_1, ar_1, 1)
    brow = jnp.concatenate([b_0, b_1]).reshape(1, 512)
    featT = _featT(x, Wcat, din=din)
    rescat, elerT = _aux(x, Wcat, rWcat, AB0, AB1, brow, din=din)
    z4 = jnp.zeros((4 * N,), jnp.float32)
    z128 = jnp.zeros((N, 128), jnp.float32)
    denP, eeT = _scores(src0, dst0, src1, dst1, elerT.reshape(16 * N), z4)
    den2 = _dred(denP.reshape(2, 32, 4 * N)).reshape(8 * N)
    rstT = _agg(src0, dst0, src1, dst1, featT, den2, eeT, z128)
    return _combine(rstT.reshape(2, 2, N, 128), rescat, act=act)


def kernel(x, edge_index_rel0, edge_index_rel1, W1_0, al1_0, ar1_0, resW1_0, b1_0, W1_1, al1_1, ar1_1, resW1_1, b1_1, W2_0, al2_0, ar2_0, resW2_0, b2_0, W2_1, al2_1, ar2_1, resW2_1, b2_1):
    src0, dst0 = edge_index_rel0[0], edge_index_rel0[1]
    src1, dst1 = edge_index_rel1[0], edge_index_rel1[1]
    h = _layer(x, src0, dst0, src1, dst1, W1_0, resW1_0, b1_0, W1_1, resW1_1,
               b1_1, al1_0, ar1_0, al1_1, ar1_1, din=128, act=True)
    g = _layer(h, src0, dst0, src1, dst1, W2_0, resW2_0, b2_0, W2_1, resW2_1,
               b2_1, al2_0, ar2_0, al2_1, ar2_1, din=256, act=False)
    return g
